# baseline (speedup 1.0000x reference)
import functools
import math

import jax
import jax.numpy as jnp
from jax import lax
from jax.experimental import pallas as pl
from jax.experimental.pallas import tpu as pltpu
from jax.experimental.pallas import tpu_sc as plsc

H, T, DK, BSC = 12, 2048, 64, 32
NB = T // BSC
NQ = H * T
NW = 32
QPW = NQ // NW
C = 4
NCHUNK = QPW // C
ROW = 2 * DK * BSC
L = 16


def _splat(vec, lane):
  idx = jnp.full((L,), lane, jnp.int32)
  return jnp.take_along_axis(vec, idx, axis=0, mode="promise_in_bounds")


def _lane_max(vec):
  return _splat(plsc.cummax(vec), L - 1)


def _lane_sum(vec):
  return _splat(plsc.cumsum(vec), L - 1)


def _load2(rows_v, row, off):
  y = plsc.bitcast(rows_v[row, pl.ds(off // 2, L)], jnp.bfloat16)
  return plsc.unpack(y, format=plsc.PackFormat.INTERLEAVED,
                     preferred_element_type=jnp.float32)


def _compute_chunk(q_v, rows_v, p_v, out_v):
  def score_body(qi, c2):
    qv = [q_v[qi, pl.ds(L * i, L)] for i in range(DK // L)]
    zero = jnp.zeros((L,), jnp.float32)
    s = [zero, zero, zero, zero]
    for blk in range(2):
      row = 2 * qi + blk
      for d in range(DK):
        qd = _splat(qv[d // L], d % L)
        ka, kb = _load2(rows_v, row, d * BSC)
        s[2 * blk] = s[2 * blk] + qd * ka
        s[2 * blk + 1] = s[2 * blk + 1] + qd * kb
    m = _lane_max(jnp.maximum(jnp.maximum(s[0], s[1]),
                              jnp.maximum(s[2], s[3])))
    e = [jnp.exp(x - m) for x in s]
    inv = 1.0 / _lane_sum(e[0] + e[1] + e[2] + e[3])
    for t in range(4):
      p_v[qi, pl.ds(L * t, L)] = e[t] * inv
    return c2

  def pv_body(qi, c2):
    zero = jnp.zeros((L,), jnp.float32)
    pv = [p_v[qi, pl.ds(L * t, L)] for t in range(4)]
    o = [zero, zero, zero, zero]
    for blk in range(2):
      row = 2 * qi + blk
      for j in range(BSC):
        pj = _splat(pv[2 * blk + j // L], j % L)
        voff = DK * BSC + j * DK
        va, vb = _load2(rows_v, row, voff)
        vc, vd = _load2(rows_v, row, voff + 2 * L)
        o[0] = o[0] + pj * va
        o[1] = o[1] + pj * vb
        o[2] = o[2] + pj * vc
        o[3] = o[3] + pj * vd
    for t in range(DK // L):
      out_v[qi, pl.ds(L * t, L)] = o[t]
    return c2

  lax.fori_loop(0, C, score_body, 0)
  lax.fori_loop(0, C, pv_body, 0)


def _attn_body(q_hbm, kv_hbm, gidx_hbm, out_hbm, idx0, idx1, q0, q1, rows0,
               rows1, p_v, out_v, sem0, sem1):
  wid = lax.axis_index("s") * 2 + lax.axis_index("c")
  base = wid * QPW
  idx = (idx0, idx1)
  qbuf = (q0, q1)
  rows = (rows0, rows1)
  sem = (sem0, sem1)

  def fetch_and_issue(ci, b):
    qb = base + ci * C
    pltpu.sync_copy(gidx_hbm.at[pl.ds(qb * 2, 2 * C)], idx[b])
    pltpu.sync_copy(q_hbm.at[pl.ds(qb, C)], qbuf[b])
    pltpu.async_copy(kv_hbm.at[idx[b]], rows[b], sem[b])

  fetch_and_issue(0, 0)

  def pair_body(si, carry):
    for b in range(2):
      ci = 2 * si + b
      nb = 1 - b

      @pl.when(ci + 1 < NCHUNK)
      def _():
        fetch_and_issue(ci + 1, nb)

      pltpu.make_async_copy(kv_hbm.at[idx[b]], rows[b], sem[b]).wait()
      _compute_chunk(qbuf[b], rows[b], p_v, out_v)
      pltpu.sync_copy(out_v, out_hbm.at[pl.ds(base + ci * C, C)])
    return carry

  lax.fori_loop(0, NCHUNK // 2, pair_body, 0)


_sc_attn = functools.partial(
    pl.kernel,
    out_type=jax.ShapeDtypeStruct((NQ, DK), jnp.float32),
    mesh=plsc.VectorSubcoreMesh(core_axis_name="c", subcore_axis_name="s"),
    scratch_types=[
        pltpu.VMEM((2 * C,), jnp.int32),
        pltpu.VMEM((2 * C,), jnp.int32),
        pltpu.VMEM((C, DK), jnp.float32),
        pltpu.VMEM((C, DK), jnp.float32),
        pltpu.VMEM((2 * C, ROW // 2), jnp.int32),
        pltpu.VMEM((2 * C, ROW // 2), jnp.int32),
        pltpu.VMEM((C, 2 * BSC), jnp.float32),
        pltpu.VMEM((C, DK), jnp.float32),
        pltpu.SemaphoreType.DMA,
        pltpu.SemaphoreType.DMA,
    ],
    compiler_params=pltpu.CompilerParams(needs_layout_passes=False),
)(_attn_body)


def kernel(q, k, v, BS, top2_idx):
  del BS
  kt = k.reshape(H, NB, BSC, DK).transpose(0, 1, 3, 2).reshape(H * NB,
                                                               DK * BSC)
  vt = v.reshape(H, NB, BSC, DK).reshape(H * NB, BSC * DK)
  kv = jnp.concatenate([kt, vt], axis=1)
  kv = kv.reshape(H * NB, ROW // (2 * L), 2, L).swapaxes(2, 3).reshape(
      H * NB, ROW).astype(jnp.bfloat16)
  kv = lax.bitcast_convert_type(kv.reshape(H * NB, ROW // 2, 2), jnp.int32)
  q2 = (q * (1.0 / math.sqrt(DK))).reshape(NQ, DK)
  head_off = (jnp.arange(H, dtype=jnp.int32) * NB)[:, None, None]
  gidx = (top2_idx.reshape(H, T, 2).astype(jnp.int32) + head_off).reshape(
      NQ * 2)
  out = _sc_attn(q2, kv, gidx)
  return out.reshape(1, H, T, DK)


# speedup vs baseline: 1.2127x; 1.2127x over previous
import functools
import math

import jax
import jax.numpy as jnp
from jax import lax
from jax.experimental import pallas as pl
from jax.experimental.pallas import tpu as pltpu
from jax.experimental.pallas import tpu_sc as plsc

H, T, DK, BSC = 12, 2048, 64, 32
NB = T // BSC
NQ = H * T
NW = 32
QPW = NQ // NW
C = 4
NCHUNK = QPW // C
ROW = 2 * DK * BSC
L = 16


def _splat(vec, lane):
  idx = jnp.full((L,), lane, jnp.int32)
  return jnp.take_along_axis(vec, idx, axis=0, mode="promise_in_bounds")


def _lane_max(vec):
  return _splat(plsc.cummax(vec), L - 1)


def _lane_sum(vec):
  return _splat(plsc.cumsum(vec), L - 1)


def _compute_chunk(q_v, rows_v, p_v, out_v):
  def score_body(qi, c2):
    qv = [q_v[qi, pl.ds(L * i, L)] for i in range(DK // L)]
    zero = jnp.zeros((L,), jnp.float32)
    s = [zero, zero, zero, zero]
    for blk in range(2):
      row = 2 * qi + blk
      for d in range(DK):
        qd = _splat(qv[d // L], d % L)
        off = d * BSC
        s[2 * blk] = s[2 * blk] + qd * rows_v[row, pl.ds(off, L)]
        s[2 * blk + 1] = s[2 * blk + 1] + qd * rows_v[row, pl.ds(off + L, L)]
    m = _lane_max(jnp.maximum(jnp.maximum(s[0], s[1]),
                              jnp.maximum(s[2], s[3])))
    e = [jnp.exp(x - m) for x in s]
    inv = 1.0 / _lane_sum(e[0] + e[1] + e[2] + e[3])
    for t in range(4):
      p_v[qi, pl.ds(L * t, L)] = e[t] * inv
    return c2

  def pv_body(qi, c2):
    zero = jnp.zeros((L,), jnp.float32)
    pv = [p_v[qi, pl.ds(L * t, L)] for t in range(4)]
    o = [zero, zero, zero, zero]
    for blk in range(2):
      row = 2 * qi + blk
      for j in range(BSC):
        pj = _splat(pv[2 * blk + j // L], j % L)
        voff = DK * BSC + j * DK
        for t in range(DK // L):
          o[t] = o[t] + pj * rows_v[row, pl.ds(voff + L * t, L)]
    for t in range(DK // L):
      out_v[qi, pl.ds(L * t, L)] = o[t]
    return c2

  lax.fori_loop(0, C, score_body, 0)
  lax.fori_loop(0, C, pv_body, 0)


def _attn_body(q_hbm, kv_hbm, gidx_hbm, out_hbm, idx0, idx1, q0, q1, rows0,
               rows1, p_v, o0, o1, sem0, sem1, osem0, osem1):
  wid = lax.axis_index("s") * 2 + lax.axis_index("c")
  base = wid * QPW
  idx = (idx0, idx1)
  qbuf = (q0, q1)
  rows = (rows0, rows1)
  obuf = (o0, o1)
  sem = (sem0, sem1)
  osem = (osem0, osem1)

  def fetch_and_issue(ci, b):
    qb = base + ci * C
    pltpu.sync_copy(gidx_hbm.at[pl.ds(qb * 2, 2 * C)], idx[b])
    pltpu.sync_copy(q_hbm.at[pl.ds(qb, C)], qbuf[b])
    pltpu.async_copy(kv_hbm.at[idx[b]], rows[b], sem[b])

  def out_dst(ci):
    return out_hbm.at[pl.ds(base + ci * C, C)]

  fetch_and_issue(0, 0)

  def pair_body(si, carry):
    for b in range(2):
      ci = 2 * si + b
      nb = 1 - b

      @pl.when(ci + 1 < NCHUNK)
      def _():
        fetch_and_issue(ci + 1, nb)

      pltpu.make_async_copy(kv_hbm.at[idx[b]], rows[b], sem[b]).wait()

      @pl.when(ci >= 2)
      def _():
        pltpu.make_async_copy(obuf[b], out_dst(ci - 2), osem[b]).wait()

      _compute_chunk(qbuf[b], rows[b], p_v, obuf[b])
      pltpu.async_copy(obuf[b], out_dst(ci), osem[b])
    return carry

  lax.fori_loop(0, NCHUNK // 2, pair_body, 0)
  pltpu.make_async_copy(obuf[0], out_dst(NCHUNK - 2), osem[0]).wait()
  pltpu.make_async_copy(obuf[1], out_dst(NCHUNK - 1), osem[1]).wait()


_sc_attn = functools.partial(
    pl.kernel,
    out_type=jax.ShapeDtypeStruct((NQ, DK), jnp.float32),
    mesh=plsc.VectorSubcoreMesh(core_axis_name="c", subcore_axis_name="s"),
    scratch_types=[
        pltpu.VMEM((2 * C,), jnp.int32),
        pltpu.VMEM((2 * C,), jnp.int32),
        pltpu.VMEM((C, DK), jnp.float32),
        pltpu.VMEM((C, DK), jnp.float32),
        pltpu.VMEM((2 * C, ROW), jnp.float32),
        pltpu.VMEM((2 * C, ROW), jnp.float32),
        pltpu.VMEM((C, 2 * BSC), jnp.float32),
        pltpu.VMEM((C, DK), jnp.float32),
        pltpu.VMEM((C, DK), jnp.float32),
        pltpu.SemaphoreType.DMA,
        pltpu.SemaphoreType.DMA,
        pltpu.SemaphoreType.DMA,
        pltpu.SemaphoreType.DMA,
    ],
    compiler_params=pltpu.CompilerParams(needs_layout_passes=False),
)(_attn_body)


def kernel(q, k, v, BS, top2_idx):
  del BS
  kt = k.reshape(H, NB, BSC, DK).transpose(0, 1, 3, 2).reshape(H * NB,
                                                               DK * BSC)
  vt = v.reshape(H, NB, BSC, DK).reshape(H * NB, BSC * DK)
  kv = jnp.concatenate([kt, vt], axis=1)
  q2 = (q * (1.0 / math.sqrt(DK))).reshape(NQ, DK)
  head_off = (jnp.arange(H, dtype=jnp.int32) * NB)[:, None, None]
  gidx = (top2_idx.reshape(H, T, 2).astype(jnp.int32) + head_off).reshape(
      NQ * 2)
  out = _sc_attn(q2, kv, gidx)
  return out.reshape(1, H, T, DK)


# speedup vs baseline: 1.5608x; 1.2870x over previous
import functools
import math

import jax
import jax.numpy as jnp
from jax import lax
from jax.experimental import pallas as pl
from jax.experimental.pallas import tpu as pltpu
from jax.experimental.pallas import tpu_sc as plsc

H, T, DK, BSC = 12, 2048, 64, 32
NB = T // BSC
NQ = H * T
NW = 32
QPW = NQ // NW
C = 4
NCHUNK = QPW // C
ROW = 2 * DK * BSC
L = 16


def _splat(vec, lane):
  idx = jnp.full((L,), lane, jnp.int32)
  return jnp.take_along_axis(vec, idx, axis=0, mode="promise_in_bounds")


def _lane_max(vec):
  return _splat(plsc.cummax(vec), L - 1)


def _lane_sum(vec):
  return _splat(plsc.cumsum(vec), L - 1)


def _compute_chunk(q_v, rows_v, p_v, out_v):
  def score_body(qi, c2):
    qv = [q_v[qi, pl.ds(L * i, L)] for i in range(DK // L)]
    zero = jnp.zeros((L,), jnp.float32)
    s = [zero, zero, zero, zero]
    for blk in range(2):
      row = 2 * qi + blk
      for d in range(DK):
        qd = _splat(qv[d // L], d % L)
        off = d * BSC
        s[2 * blk] = s[2 * blk] + qd * rows_v[row, pl.ds(off, L)]
        s[2 * blk + 1] = s[2 * blk + 1] + qd * rows_v[row, pl.ds(off + L, L)]
    m = _lane_max(jnp.maximum(jnp.maximum(s[0], s[1]),
                              jnp.maximum(s[2], s[3])))
    e = [jnp.exp(x - m) for x in s]
    inv = 1.0 / _lane_sum(e[0] + e[1] + e[2] + e[3])
    for t in range(4):
      p_v[qi, pl.ds(L * t, L)] = e[t] * inv
    return c2

  def pv_body(qi, c2):
    zero = jnp.zeros((L,), jnp.float32)
    pv = [p_v[qi, pl.ds(L * t, L)] for t in range(4)]
    o = [zero, zero, zero, zero]
    for blk in range(2):
      row = 2 * qi + blk
      for j in range(BSC):
        pj = _splat(pv[2 * blk + j // L], j % L)
        voff = DK * BSC + j * DK
        for t in range(DK // L):
          o[t] = o[t] + pj * rows_v[row, pl.ds(voff + L * t, L)]
    for t in range(DK // L):
      out_v[qi, pl.ds(L * t, L)] = o[t]
    return c2

  lax.fori_loop(0, C, score_body, 0)
  lax.fori_loop(0, C, pv_body, 0)


def _attn_body(q_hbm, kv_hbm, gidx_hbm, out_hbm, idx0, idx1, q0, q1, rows0,
               rows1, p_v, o0, o1, sem0, sem1, isem0, isem1, qsem0, qsem1,
               osem0, osem1):
  wid = lax.axis_index("s") * 2 + lax.axis_index("c")
  base = wid * QPW
  idx = (idx0, idx1)
  qbuf = (q0, q1)
  rows = (rows0, rows1)
  obuf = (o0, o1)
  sem = (sem0, sem1)
  isem = (isem0, isem1)
  qsem = (qsem0, qsem1)
  osem = (osem0, osem1)

  def idx_copy(ci, b):
    return pltpu.make_async_copy(
        gidx_hbm.at[pl.ds((base + ci * C) * 2, 2 * C)], idx[b], isem[b])

  def q_copy(ci, b):
    return pltpu.make_async_copy(q_hbm.at[pl.ds(base + ci * C, C)], qbuf[b],
                                 qsem[b])

  def out_dst(ci):
    return out_hbm.at[pl.ds(base + ci * C, C)]

  idx_copy(0, 0).start()
  idx_copy(1, 1).start()
  q_copy(0, 0).start()
  idx_copy(0, 0).wait()
  pltpu.async_copy(kv_hbm.at[idx[0]], rows[0], sem[0])

  def pair_body(si, carry):
    for b in range(2):
      ci = 2 * si + b
      nb = 1 - b

      @pl.when(ci + 1 < NCHUNK)
      def _():
        idx_copy(ci + 1, nb).wait()
        pltpu.async_copy(kv_hbm.at[idx[nb]], rows[nb], sem[nb])

      pltpu.make_async_copy(kv_hbm.at[idx[b]], rows[b], sem[b]).wait()

      @pl.when(ci + 2 < NCHUNK)
      def _():
        idx_copy(ci + 2, b).start()

      @pl.when(ci + 1 < NCHUNK)
      def _():
        q_copy(ci + 1, nb).start()

      q_copy(ci, b).wait()

      @pl.when(ci >= 2)
      def _():
        pltpu.make_async_copy(obuf[b], out_dst(ci - 2), osem[b]).wait()

      _compute_chunk(qbuf[b], rows[b], p_v, obuf[b])
      pltpu.async_copy(obuf[b], out_dst(ci), osem[b])
    return carry

  lax.fori_loop(0, NCHUNK // 2, pair_body, 0)
  pltpu.make_async_copy(obuf[0], out_dst(NCHUNK - 2), osem[0]).wait()
  pltpu.make_async_copy(obuf[1], out_dst(NCHUNK - 1), osem[1]).wait()


_sc_attn = functools.partial(
    pl.kernel,
    out_type=jax.ShapeDtypeStruct((NQ, DK), jnp.float32),
    mesh=plsc.VectorSubcoreMesh(core_axis_name="c", subcore_axis_name="s"),
    scratch_types=[
        pltpu.VMEM((2 * C,), jnp.int32),
        pltpu.VMEM((2 * C,), jnp.int32),
        pltpu.VMEM((C, DK), jnp.float32),
        pltpu.VMEM((C, DK), jnp.float32),
        pltpu.VMEM((2 * C, ROW), jnp.float32),
        pltpu.VMEM((2 * C, ROW), jnp.float32),
        pltpu.VMEM((C, 2 * BSC), jnp.float32),
        pltpu.VMEM((C, DK), jnp.float32),
        pltpu.VMEM((C, DK), jnp.float32),
        pltpu.SemaphoreType.DMA,
        pltpu.SemaphoreType.DMA,
        pltpu.SemaphoreType.DMA,
        pltpu.SemaphoreType.DMA,
        pltpu.SemaphoreType.DMA,
        pltpu.SemaphoreType.DMA,
        pltpu.SemaphoreType.DMA,
        pltpu.SemaphoreType.DMA,
    ],
    compiler_params=pltpu.CompilerParams(needs_layout_passes=False),
)(_attn_body)


def kernel(q, k, v, BS, top2_idx):
  del BS
  kt = k.reshape(H, NB, BSC, DK).transpose(0, 1, 3, 2).reshape(H * NB,
                                                               DK * BSC)
  vt = v.reshape(H, NB, BSC, DK).reshape(H * NB, BSC * DK)
  kv = jnp.concatenate([kt, vt], axis=1)
  q2 = (q * (1.0 / math.sqrt(DK))).reshape(NQ, DK)
  head_off = (jnp.arange(H, dtype=jnp.int32) * NB)[:, None, None]
  gidx = (top2_idx.reshape(H, T, 2).astype(jnp.int32) + head_off).reshape(
      NQ * 2)
  out = _sc_attn(q2, kv, gidx)
  return out.reshape(1, H, T, DK)


# speedup vs baseline: 1.5768x; 1.0102x over previous
import functools
import math

import jax
import jax.numpy as jnp
from jax import lax
from jax.experimental import pallas as pl
from jax.experimental.pallas import tpu as pltpu
from jax.experimental.pallas import tpu_sc as plsc

H, T, DK, BSC = 12, 2048, 64, 32
NB = T // BSC
NQ = H * T
NW = 32
QPW = NQ // NW
C = 4
NCHUNK = QPW // C
ROW = 2 * DK * BSC
L = 16


def _splat(vec, lane):
  idx = jnp.full((L,), lane, jnp.int32)
  return jnp.take_along_axis(vec, idx, axis=0, mode="promise_in_bounds")


def _lane_max(vec):
  return _splat(plsc.cummax(vec), L - 1)


def _lane_sum(vec):
  return _splat(plsc.cumsum(vec), L - 1)


def _compute_chunk(q_v, rk_v, rv_v, p_v, out_v):
  def score_body(qi, c2):
    qv = [q_v[qi, pl.ds(L * i, L)] for i in range(DK // L)]
    zero = jnp.zeros((L,), jnp.float32)
    s = [zero, zero, zero, zero]
    for blk in range(2):
      row = 2 * qi + blk
      for d in range(DK):
        qd = _splat(qv[d // L], d % L)
        off = d * BSC
        s[2 * blk] = s[2 * blk] + qd * rk_v[row, pl.ds(off, L)]
        s[2 * blk + 1] = s[2 * blk + 1] + qd * rk_v[row, pl.ds(off + L, L)]
    m = _lane_max(jnp.maximum(jnp.maximum(s[0], s[1]),
                              jnp.maximum(s[2], s[3])))
    e = [jnp.exp(x - m) for x in s]
    inv = 1.0 / _lane_sum(e[0] + e[1] + e[2] + e[3])
    for t in range(4):
      p_v[qi, pl.ds(L * t, L)] = e[t] * inv
    return c2

  def pv_body(qi, c2):
    zero = jnp.zeros((L,), jnp.float32)
    pv = [p_v[qi, pl.ds(L * t, L)] for t in range(4)]
    o = [zero, zero, zero, zero]
    for blk in range(2):
      row = 2 * qi + blk
      for j in range(BSC):
        pj = _splat(pv[2 * blk + j // L], j % L)
        voff = j * DK
        for t in range(DK // L):
          o[t] = o[t] + pj * rv_v[row, pl.ds(voff + L * t, L)]
    for t in range(DK // L):
      out_v[qi, pl.ds(L * t, L)] = o[t]
    return c2

  lax.fori_loop(0, C, score_body, 0)
  lax.fori_loop(0, C, pv_body, 0)


def _attn_body(q_hbm, kt_hbm, vt_hbm, gidx_hbm, out_hbm, idx0, idx1, q0, q1,
               rk0, rk1, rv0, rv1, p_v, o0, o1, sem0, sem1, vsem0, vsem1,
               isem0, isem1, qsem0, qsem1, osem0, osem1):
  wid = lax.axis_index("s") * 2 + lax.axis_index("c")
  base = wid * QPW
  idx = (idx0, idx1)
  qbuf = (q0, q1)
  rk = (rk0, rk1)
  rv = (rv0, rv1)
  obuf = (o0, o1)
  sem = (sem0, sem1)
  vsem = (vsem0, vsem1)
  isem = (isem0, isem1)
  qsem = (qsem0, qsem1)
  osem = (osem0, osem1)

  def idx_copy(ci, b):
    return pltpu.make_async_copy(
        gidx_hbm.at[pl.ds((base + ci * C) * 2, 2 * C)], idx[b], isem[b])

  def q_copy(ci, b):
    return pltpu.make_async_copy(q_hbm.at[pl.ds(base + ci * C, C)], qbuf[b],
                                 qsem[b])

  def out_dst(ci):
    return out_hbm.at[pl.ds(base + ci * C, C)]

  idx_copy(0, 0).start()
  idx_copy(1, 1).start()
  q_copy(0, 0).start()
  idx_copy(0, 0).wait()
  pltpu.async_copy(kt_hbm.at[idx[0]], rk[0], sem[0])
  pltpu.async_copy(vt_hbm.at[idx[0]], rv[0], vsem[0])

  def pair_body(si, carry):
    for b in range(2):
      ci = 2 * si + b
      nb = 1 - b

      @pl.when(ci + 1 < NCHUNK)
      def _():
        idx_copy(ci + 1, nb).wait()
        pltpu.async_copy(kt_hbm.at[idx[nb]], rk[nb], sem[nb])
        pltpu.async_copy(vt_hbm.at[idx[nb]], rv[nb], vsem[nb])

      pltpu.make_async_copy(kt_hbm.at[idx[b]], rk[b], sem[b]).wait()
      pltpu.make_async_copy(vt_hbm.at[idx[b]], rv[b], vsem[b]).wait()

      @pl.when(ci + 2 < NCHUNK)
      def _():
        idx_copy(ci + 2, b).start()

      @pl.when(ci + 1 < NCHUNK)
      def _():
        q_copy(ci + 1, nb).start()

      q_copy(ci, b).wait()

      @pl.when(ci >= 2)
      def _():
        pltpu.make_async_copy(obuf[b], out_dst(ci - 2), osem[b]).wait()

      _compute_chunk(qbuf[b], rk[b], rv[b], p_v, obuf[b])
      pltpu.async_copy(obuf[b], out_dst(ci), osem[b])
    return carry

  lax.fori_loop(0, NCHUNK // 2, pair_body, 0)
  pltpu.make_async_copy(obuf[0], out_dst(NCHUNK - 2), osem[0]).wait()
  pltpu.make_async_copy(obuf[1], out_dst(NCHUNK - 1), osem[1]).wait()


_sc_attn = functools.partial(
    pl.kernel,
    out_type=jax.ShapeDtypeStruct((NQ, DK), jnp.float32),
    mesh=plsc.VectorSubcoreMesh(core_axis_name="c", subcore_axis_name="s"),
    scratch_types=[
        pltpu.VMEM((2 * C,), jnp.int32),
        pltpu.VMEM((2 * C,), jnp.int32),
        pltpu.VMEM((C, DK), jnp.float32),
        pltpu.VMEM((C, DK), jnp.float32),
        pltpu.VMEM((2 * C, ROW // 2), jnp.float32),
        pltpu.VMEM((2 * C, ROW // 2), jnp.float32),
        pltpu.VMEM((2 * C, ROW // 2), jnp.float32),
        pltpu.VMEM((2 * C, ROW // 2), jnp.float32),
        pltpu.VMEM((C, 2 * BSC), jnp.float32),
        pltpu.VMEM((C, DK), jnp.float32),
        pltpu.VMEM((C, DK), jnp.float32),
        pltpu.SemaphoreType.DMA,
        pltpu.SemaphoreType.DMA,
        pltpu.SemaphoreType.DMA,
        pltpu.SemaphoreType.DMA,
        pltpu.SemaphoreType.DMA,
        pltpu.SemaphoreType.DMA,
        pltpu.SemaphoreType.DMA,
        pltpu.SemaphoreType.DMA,
        pltpu.SemaphoreType.DMA,
        pltpu.SemaphoreType.DMA,
    ],
    compiler_params=pltpu.CompilerParams(needs_layout_passes=False),
)(_attn_body)


def kernel(q, k, v, BS, top2_idx):
  del BS
  kt = (k * (1.0 / math.sqrt(DK))).reshape(H, NB, BSC, DK).transpose(
      0, 1, 3, 2).reshape(H * NB, DK * BSC)
  vt = v.reshape(H * NB, BSC * DK)
  q2 = q.reshape(NQ, DK)
  head_off = (jnp.arange(H, dtype=jnp.int32) * NB)[:, None, None]
  gidx = (top2_idx.reshape(H, T, 2).astype(jnp.int32) + head_off).reshape(
      NQ * 2)
  out = _sc_attn(q2, kt, vt, gidx)
  return out.reshape(1, H, T, DK)
